# baseline (speedup 1.0000x reference)
import jax
import jax.numpy as jnp
from jax import lax
from jax.experimental import pallas as pl
from jax.experimental.pallas import tpu as pltpu
from jax.experimental.pallas import tpu_sc as plsc

B = 32
NCH = 3
H = 512
W = 512
NPIX = H * W
NBINS = 64
LANES = 16
NC, NS = 2, 16
UNROLL = 4
NPIXW = NPIX // 2
CHUNK = 8192
NSTEPS = NPIXW // CHUNK
NVREG = CHUNK // LANES
HSIZE = NBINS * LANES
LAMBDA_CC = 10.0
EPS = 1e-06


KSPLIT = 4
HS = H // KSPLIT


def _tc_bin_body(w_ref, x_ref, y_ref, ix_ref, iy_ref, s_ref):
    k = pl.program_id(1)
    w0 = w_ref[0]
    w1 = w_ref[1]
    w2 = w_ref[2]
    w3 = w_ref[3]
    xr = x_ref[0, 0]
    xg = x_ref[0, 1]
    xb = x_ref[0, 2]
    yr = y_ref[0, 0]
    yg = y_ref[0, 1]
    yb = y_ref[0, 2]
    cx = xr * w0 + xg * w1 + xb * w2 + w3
    cy = yr * w0 + yg * w1 + yb * w2 + w3
    ix = jnp.minimum(jnp.maximum(cx, 0.0), 63.0).astype(jnp.int32)
    iy = jnp.minimum(jnp.maximum(cy, 0.0), 63.0).astype(jnp.int32)
    ix_ref[0] = ix[: HS // 2] | (ix[HS // 2:] << 16)
    iy_ref[0] = iy[: HS // 2] | (iy[HS // 2:] << 16)
    zz = jnp.zeros((1, 1), jnp.float32)
    part = jnp.concatenate(
        [jnp.sum(p, keepdims=True) for p in (xr, xg, xb, yr, yg, yb)]
        + [zz, zz], axis=1).reshape(1, 1, 8)

    @pl.when(k == 0)
    def _():
        s_ref[...] = part

    @pl.when(k > 0)
    def _():
        s_ref[...] = s_ref[...] + part


def _tc_bin(x, y, wq):
    return pl.pallas_call(
        _tc_bin_body,
        grid=(B, KSPLIT),
        in_specs=[
            pl.BlockSpec(memory_space=pltpu.SMEM),
            pl.BlockSpec((1, NCH, HS, W), lambda b, k: (b, 0, k, 0)),
            pl.BlockSpec((1, NCH, HS, W), lambda b, k: (b, 0, k, 0)),
        ],
        out_specs=[
            pl.BlockSpec((1, HS // 2, W), lambda b, k: (b, k, 0)),
            pl.BlockSpec((1, HS // 2, W), lambda b, k: (b, k, 0)),
            pl.BlockSpec((1, 1, 8), lambda b, k: (b, 0, 0)),
        ],
        out_shape=[
            jax.ShapeDtypeStruct((B, H // 2, W), jnp.int32),
            jax.ShapeDtypeStruct((B, H // 2, W), jnp.int32),
            jax.ShapeDtypeStruct((B, 1, 8), jnp.float32),
        ],
    )(wq, x, y)


def _sc_hist_body(x_hbm, y_hbm, hx_out, hy_out, xbuf, ybuf, hx, hy,
                  sem0, sem1):
    img = lax.axis_index("c") * NS + lax.axis_index("s")
    sems = (sem0, sem1)
    li = lax.iota(jnp.int32, LANES)
    ones = jnp.full((LANES,), 1.0, jnp.float32)
    zeros = jnp.zeros((LANES,), jnp.float32)

    for i in range(NBINS):
        hx[pl.ds(i * LANES, LANES)] = zeros
        hy[pl.ds(i * LANES, LANES)] = zeros

    xb32 = xbuf
    yb32 = ybuf

    def start(s, b):
        pltpu.async_copy(
            x_hbm.at[pl.ds(img * NPIXW + s * CHUNK, CHUNK)],
            xbuf.at[pl.ds(b * CHUNK, CHUNK)], sems[b])
        pltpu.async_copy(
            y_hbm.at[pl.ds(img * NPIXW + s * CHUNK, CHUNK)],
            ybuf.at[pl.ds(b * CHUNK, CHUNK)], sems[b])

    def wait(b):
        pltpu.make_async_copy(
            x_hbm.at[pl.ds(0, CHUNK)],
            xbuf.at[pl.ds(b * CHUNK, CHUNK)], sems[b]).wait()
        pltpu.make_async_copy(
            y_hbm.at[pl.ds(0, CHUNK)],
            ybuf.at[pl.ds(b * CHUNK, CHUNK)], sems[b]).wait()

    def scatter2(h, w):
        lo = ((w & 63) << 4) | li
        hi = ((w >> 16) & 63) << 4 | li
        plsc.addupdate_scatter(h, [lo], ones)
        plsc.addupdate_scatter(h, [hi], ones)

    def chunk_compute(b):
        base = b * CHUNK

        def jbody(j, carry):
            o = j * LANES
            scatter2(hx, xb32[pl.ds(base + o, LANES)])
            scatter2(hy, yb32[pl.ds(base + o, LANES)])
            return carry

        return plsc.parallel_loop(0, NVREG, 1, unroll=UNROLL,
                                  carry=jnp.int32(0))(jbody)

    start(0, 0)

    def outer(i, carry):
        s0 = 2 * i
        start(s0 + 1, 1)
        wait(0)
        chunk_compute(0)

        @pl.when(i < NSTEPS // 2 - 1)
        def _():
            start(s0 + 2, 0)

        wait(1)
        chunk_compute(1)
        return carry

    lax.fori_loop(0, NSTEPS // 2, outer, jnp.int32(0))

    pltpu.sync_copy(hx, hx_out.at[pl.ds(img * HSIZE, HSIZE)])
    pltpu.sync_copy(hy, hy_out.at[pl.ds(img * HSIZE, HSIZE)])


def _sc_hist(ixf, iyf):
    mesh = plsc.VectorSubcoreMesh(core_axis_name="c", subcore_axis_name="s",
                                  num_cores=NC, num_subcores=NS)
    f = pl.kernel(
        _sc_hist_body,
        out_type=(
            jax.ShapeDtypeStruct((B * HSIZE,), jnp.float32),
            jax.ShapeDtypeStruct((B * HSIZE,), jnp.float32),
        ),
        mesh=mesh,
        scratch_types=[
            pltpu.VMEM((2 * CHUNK,), jnp.int32),
            pltpu.VMEM((2 * CHUNK,), jnp.int32),
            pltpu.VMEM((HSIZE,), jnp.float32),
            pltpu.VMEM((HSIZE,), jnp.float32),
            pltpu.SemaphoreType.DMA,
            pltpu.SemaphoreType.DMA,
        ],
        compiler_params=pltpu.CompilerParams(needs_layout_passes=False),
        name="cc_hist_sc",
    )
    return f(ixf, iyf)


def _tc_final_body(hx_ref, hy_ref, s_ref, out_ref):
    hx = 2.0 * jnp.sum(hx_ref[...], axis=2)
    hy = 2.0 * jnp.sum(hy_ref[...], axis=2)
    s = s_ref[...]
    m01 = (s * (1.0 / NPIX) + 1.0) * 0.5
    xm = m01[:, 0:3]
    ym = m01[:, 3:6]
    rw = xm[:, 0:1]
    gw = xm[:, 1:2]
    bw = xm[:, 2:3]
    grey_world = jnp.mean(jnp.abs(rw - gw) + jnp.abs(gw - bw) + jnp.abs(bw - rw))
    x_ratio = xm / (jnp.sum(xm, axis=1, keepdims=True) + EPS)
    y_ratio = ym / (jnp.sum(ym, axis=1, keepdims=True) + EPS)
    ratio_loss = jnp.mean(jnp.abs(x_ratio - y_ratio))
    log_x = jnp.log(hx)
    kl_pt = jnp.where(hy > 0,
                      hy * (jnp.log(jnp.where(hy > 0, hy, 1.0)) - log_x), 0.0)
    kl_div = jnp.sum(kl_pt) / B
    out_ref[...] = jnp.full((1, 1), LAMBDA_CC * (grey_world + ratio_loss + kl_div),
                            jnp.float32)


def _tc_final(hx, hy, s):
    return pl.pallas_call(
        _tc_final_body,
        out_shape=jax.ShapeDtypeStruct((1, 1), jnp.float32),
    )(hx, hy, s)


def kernel(x, y, lum_w, bin_edges):
    lw = lum_w.reshape(NCH)
    nb = bin_edges.shape[0] - 1
    scale = nb / (bin_edges[-1] - bin_edges[0])
    wrows = 0.5 * scale * lw
    woff = scale * (0.5 * jnp.sum(lw) - bin_edges[0])
    wq = jnp.concatenate([wrows, woff[None]], axis=0)
    ix3, iy3, sums = _tc_bin(x, y, wq)
    hxf, hyf = _sc_hist(ix3.reshape(B * NPIXW), iy3.reshape(B * NPIXW))
    out = _tc_final(hxf.reshape(B, NBINS, LANES),
                    hyf.reshape(B, NBINS, LANES), sums.reshape(B, 8))
    return out.reshape(())


# speedup vs baseline: 1.2878x; 1.2878x over previous
import jax
import jax.numpy as jnp
from jax import lax
from jax.experimental import pallas as pl
from jax.experimental.pallas import tpu as pltpu
from jax.experimental.pallas import tpu_sc as plsc

B = 32
NCH = 3
H = 512
W = 512
NPIX = H * W
NBINS = 64
LANES = 16
NC, NS = 2, 16
UNROLL = 4
NPIXW = NPIX // 2
CHUNK = 8192
NSTEPS = NPIXW // CHUNK
NVREG = CHUNK // LANES
HSIZE = NBINS * LANES
LAMBDA_CC = 10.0
EPS = 1e-06


KSPLIT = 1
HS = H // KSPLIT


def _tc_bin_body(w_ref, x_ref, y_ref, ix_ref, iy_ref, s_ref):
    k = pl.program_id(1)
    w0 = w_ref[0]
    w1 = w_ref[1]
    w2 = w_ref[2]
    w3 = w_ref[3]
    xr = x_ref[0, 0]
    xg = x_ref[0, 1]
    xb = x_ref[0, 2]
    yr = y_ref[0, 0]
    yg = y_ref[0, 1]
    yb = y_ref[0, 2]
    cx = xr * w0 + xg * w1 + xb * w2 + w3
    cy = yr * w0 + yg * w1 + yb * w2 + w3
    ix = jnp.minimum(jnp.maximum(cx, 0.0), 63.0).astype(jnp.int32)
    iy = jnp.minimum(jnp.maximum(cy, 0.0), 63.0).astype(jnp.int32)
    ix_ref[0] = ix[: HS // 2] | (ix[HS // 2:] << 16)
    iy_ref[0] = iy[: HS // 2] | (iy[HS // 2:] << 16)
    zz = jnp.zeros((1, 1), jnp.float32)
    part = jnp.concatenate(
        [jnp.sum(p, keepdims=True) for p in (xr, xg, xb, yr, yg, yb)]
        + [zz, zz], axis=1).reshape(1, 1, 8)

    @pl.when(k == 0)
    def _():
        s_ref[...] = part

    @pl.when(k > 0)
    def _():
        s_ref[...] = s_ref[...] + part


def _tc_bin(x, y, wq):
    return pl.pallas_call(
        _tc_bin_body,
        grid=(B, KSPLIT),
        in_specs=[
            pl.BlockSpec(memory_space=pltpu.SMEM),
            pl.BlockSpec((1, NCH, HS, W), lambda b, k: (b, 0, k, 0)),
            pl.BlockSpec((1, NCH, HS, W), lambda b, k: (b, 0, k, 0)),
        ],
        out_specs=[
            pl.BlockSpec((1, HS // 2, W), lambda b, k: (b, k, 0)),
            pl.BlockSpec((1, HS // 2, W), lambda b, k: (b, k, 0)),
            pl.BlockSpec((1, 1, 8), lambda b, k: (b, 0, 0)),
        ],
        out_shape=[
            jax.ShapeDtypeStruct((B, H // 2, W), jnp.int32),
            jax.ShapeDtypeStruct((B, H // 2, W), jnp.int32),
            jax.ShapeDtypeStruct((B, 1, 8), jnp.float32),
        ],
    )(wq, x, y)


def _sc_hist_body(x_hbm, y_hbm, hx_out, hy_out, xbuf, ybuf, hx, hy,
                  sem0, sem1):
    img = lax.axis_index("c") * NS + lax.axis_index("s")
    sems = (sem0, sem1)
    li = lax.iota(jnp.int32, LANES)
    ones = jnp.full((LANES,), 1.0, jnp.float32)
    zeros = jnp.zeros((LANES,), jnp.float32)

    for i in range(NBINS):
        hx[pl.ds(i * LANES, LANES)] = zeros
        hy[pl.ds(i * LANES, LANES)] = zeros

    xb32 = xbuf
    yb32 = ybuf

    def start(s, b):
        pltpu.async_copy(
            x_hbm.at[pl.ds(img * NPIXW + s * CHUNK, CHUNK)],
            xbuf.at[pl.ds(b * CHUNK, CHUNK)], sems[b])
        pltpu.async_copy(
            y_hbm.at[pl.ds(img * NPIXW + s * CHUNK, CHUNK)],
            ybuf.at[pl.ds(b * CHUNK, CHUNK)], sems[b])

    def wait(b):
        pltpu.make_async_copy(
            x_hbm.at[pl.ds(0, CHUNK)],
            xbuf.at[pl.ds(b * CHUNK, CHUNK)], sems[b]).wait()
        pltpu.make_async_copy(
            y_hbm.at[pl.ds(0, CHUNK)],
            ybuf.at[pl.ds(b * CHUNK, CHUNK)], sems[b]).wait()

    def scatter2(h, w):
        lo = ((w & 63) << 4) | li
        hi = ((w >> 16) & 63) << 4 | li
        plsc.addupdate_scatter(h, [lo], ones)
        plsc.addupdate_scatter(h, [hi], ones)

    def chunk_compute(b):
        base = b * CHUNK

        def jbody(j, carry):
            o = j * LANES
            scatter2(hx, xb32[pl.ds(base + o, LANES)])
            scatter2(hy, yb32[pl.ds(base + o, LANES)])
            return carry

        return plsc.parallel_loop(0, NVREG, 1, unroll=UNROLL,
                                  carry=jnp.int32(0))(jbody)

    start(0, 0)

    def outer(i, carry):
        s0 = 2 * i
        start(s0 + 1, 1)
        wait(0)
        chunk_compute(0)

        @pl.when(i < NSTEPS // 2 - 1)
        def _():
            start(s0 + 2, 0)

        wait(1)
        chunk_compute(1)
        return carry

    lax.fori_loop(0, NSTEPS // 2, outer, jnp.int32(0))

    pltpu.sync_copy(hx, hx_out.at[pl.ds(img * HSIZE, HSIZE)])
    pltpu.sync_copy(hy, hy_out.at[pl.ds(img * HSIZE, HSIZE)])


def _sc_hist(ixf, iyf):
    mesh = plsc.VectorSubcoreMesh(core_axis_name="c", subcore_axis_name="s",
                                  num_cores=NC, num_subcores=NS)
    f = pl.kernel(
        _sc_hist_body,
        out_type=(
            jax.ShapeDtypeStruct((B * HSIZE,), jnp.float32),
            jax.ShapeDtypeStruct((B * HSIZE,), jnp.float32),
        ),
        mesh=mesh,
        scratch_types=[
            pltpu.VMEM((2 * CHUNK,), jnp.int32),
            pltpu.VMEM((2 * CHUNK,), jnp.int32),
            pltpu.VMEM((HSIZE,), jnp.float32),
            pltpu.VMEM((HSIZE,), jnp.float32),
            pltpu.SemaphoreType.DMA,
            pltpu.SemaphoreType.DMA,
        ],
        compiler_params=pltpu.CompilerParams(needs_layout_passes=False),
        name="cc_hist_sc",
    )
    return f(ixf, iyf)


def _tc_final_body(hx_ref, hy_ref, s_ref, out_ref):
    hx = 2.0 * jnp.sum(hx_ref[...], axis=2)
    hy = 2.0 * jnp.sum(hy_ref[...], axis=2)
    s = s_ref[...]
    m01 = (s * (1.0 / NPIX) + 1.0) * 0.5
    xm = m01[:, 0:3]
    ym = m01[:, 3:6]
    rw = xm[:, 0:1]
    gw = xm[:, 1:2]
    bw = xm[:, 2:3]
    grey_world = jnp.mean(jnp.abs(rw - gw) + jnp.abs(gw - bw) + jnp.abs(bw - rw))
    x_ratio = xm / (jnp.sum(xm, axis=1, keepdims=True) + EPS)
    y_ratio = ym / (jnp.sum(ym, axis=1, keepdims=True) + EPS)
    ratio_loss = jnp.mean(jnp.abs(x_ratio - y_ratio))
    log_x = jnp.log(hx)
    kl_pt = jnp.where(hy > 0,
                      hy * (jnp.log(jnp.where(hy > 0, hy, 1.0)) - log_x), 0.0)
    kl_div = jnp.sum(kl_pt) / B
    out_ref[...] = jnp.full((1, 1), LAMBDA_CC * (grey_world + ratio_loss + kl_div),
                            jnp.float32)


def _tc_final(hx, hy, s):
    return pl.pallas_call(
        _tc_final_body,
        out_shape=jax.ShapeDtypeStruct((1, 1), jnp.float32),
    )(hx, hy, s)


def kernel(x, y, lum_w, bin_edges):
    lw = lum_w.reshape(NCH)
    nb = bin_edges.shape[0] - 1
    scale = nb / (bin_edges[-1] - bin_edges[0])
    wrows = 0.5 * scale * lw
    woff = scale * (0.5 * jnp.sum(lw) - bin_edges[0])
    wq = jnp.concatenate([wrows, woff[None]], axis=0)
    ix3, iy3, sums = _tc_bin(x, y, wq)
    hxf, hyf = _sc_hist(ix3.reshape(B * NPIXW), iy3.reshape(B * NPIXW))
    out = _tc_final(hxf.reshape(B, NBINS, LANES),
                    hyf.reshape(B, NBINS, LANES), sums.reshape(B, 8))
    return out.reshape(())
